# baseline (speedup 1.0000x reference)
import functools

import jax
import jax.numpy as jnp
from jax import lax
from jax.experimental import pallas as pl
from jax.experimental.pallas import tpu as pltpu
from jax.experimental.pallas import tpu_sc as plsc


def kernel(x, pos_table):
    B, S, D = x.shape
    NC, NS = 2, 16
    NW = NC * NS
    ROWS_W = S // NW
    CH = 16
    NCH = ROWS_W // CH
    CHW = CH * D

    xf = x.reshape(-1)
    pf = pos_table[:S].reshape(-1)
    mesh = plsc.VectorSubcoreMesh(core_axis_name="c", subcore_axis_name="s")

    @functools.partial(
        pl.kernel,
        out_type=jax.ShapeDtypeStruct((B * S * D,), jnp.float32),
        mesh=mesh,
        scratch_types=[
            pltpu.VMEM((CHW,), jnp.float32),
            pltpu.VMEM((CHW,), jnp.float32),
        ],
    )
    def sc_add(x_hbm, p_hbm, o_hbm, vp, vx):
        wid = lax.axis_index("s") * NC + lax.axis_index("c")
        base = wid * ROWS_W * D
        for c in range(NCH):
            pltpu.sync_copy(p_hbm.at[pl.ds(base + c * CHW, CHW)], vp)
            for b in range(B):
                off = b * S * D + base + c * CHW
                pltpu.sync_copy(x_hbm.at[pl.ds(off, CHW)], vx)

                def body(i, carry):
                    sl = pl.ds(i * 16, 16)
                    vx[sl] = vx[sl] + vp[sl]
                    return carry

                lax.fori_loop(0, CHW // 16, body, 0)
                pltpu.sync_copy(vx, o_hbm.at[pl.ds(off, CHW)])

    out = sc_add(xf, pf)
    return out.reshape(B, S, D)


# speedup vs baseline: 1.7211x; 1.7211x over previous
import functools

import jax
import jax.numpy as jnp
from jax import lax
from jax.experimental import pallas as pl
from jax.experimental.pallas import tpu as pltpu
from jax.experimental.pallas import tpu_sc as plsc


def kernel(x, pos_table):
    B, S, D = x.shape
    NC, NS = 2, 16
    NW = NC * NS
    ROWS_W = S // NW
    CH = 8
    NCH = ROWS_W // CH
    CHW = CH * D
    TOT = NCH * B

    xf = x.reshape(-1)
    pf = pos_table[:S].reshape(-1)
    mesh = plsc.VectorSubcoreMesh(core_axis_name="c", subcore_axis_name="s")

    @functools.partial(
        pl.kernel,
        out_type=jax.ShapeDtypeStruct((B * S * D,), jnp.float32),
        mesh=mesh,
        scratch_types=[
            pltpu.VMEM((CHW,), jnp.float32),
            pltpu.VMEM((CHW,), jnp.float32),
            pltpu.VMEM((CHW,), jnp.float32),
            pltpu.VMEM((CHW,), jnp.float32),
            pltpu.SemaphoreType.DMA,
            pltpu.SemaphoreType.DMA,
            pltpu.SemaphoreType.DMA,
            pltpu.SemaphoreType.DMA,
            pltpu.SemaphoreType.DMA,
            pltpu.SemaphoreType.DMA,
        ],
    )
    def sc_add(x_hbm, p_hbm, o_hbm, vp, vx0, vx1, vx2,
               si0, si1, si2, so0, so1, so2):
        bufs = (vx0, vx1, vx2)
        sin = (si0, si1, si2)
        sout = (so0, so1, so2)
        wid = lax.axis_index("s") * NC + lax.axis_index("c")
        base = wid * ROWS_W * D

        def off(t):
            c, b = divmod(t, B)
            return b * S * D + base + c * CHW

        pltpu.async_copy(x_hbm.at[pl.ds(off(0), CHW)], bufs[0], sin[0])
        pltpu.async_copy(x_hbm.at[pl.ds(off(1), CHW)], bufs[1], sin[1])

        for c in range(NCH):
            pltpu.sync_copy(p_hbm.at[pl.ds(base + c * CHW, CHW)], vp)
            for b in range(B):
                t = c * B + b
                k = t % 3
                vx = bufs[k]
                pltpu.make_async_copy(
                    x_hbm.at[pl.ds(off(t), CHW)], vx, sin[k]).wait()

                @plsc.parallel_loop(0, CHW // 16, unroll=8)
                def _(i):
                    sl = pl.ds(i * 16, 16)
                    vx[sl] = vx[sl] + vp[sl]

                pltpu.async_copy(vx, o_hbm.at[pl.ds(off(t), CHW)], sout[k])
                if t + 2 < TOT:
                    k2 = (t + 2) % 3
                    if t - 1 >= 0:
                        pltpu.make_async_copy(
                            bufs[k2], o_hbm.at[pl.ds(off(t - 1), CHW)],
                            sout[k2]).wait()
                    pltpu.async_copy(
                        x_hbm.at[pl.ds(off(t + 2), CHW)], bufs[k2], sin[k2])

        for t in range(TOT - 3, TOT):
            k = t % 3
            pltpu.make_async_copy(
                bufs[k], o_hbm.at[pl.ds(off(t), CHW)], sout[k]).wait()

    out = sc_add(xf, pf)
    return out.reshape(B, S, D)


# speedup vs baseline: 7.5708x; 4.3988x over previous
import jax
import jax.numpy as jnp
from jax.experimental import pallas as pl
from jax.experimental.pallas import tpu as pltpu


def _add_kernel(x_ref, p_ref, o_ref):
    o_ref[...] = x_ref[...] + p_ref[...]


def kernel(x, pos_table):
    B, S, D = x.shape
    SB = 1024
    grid = (S // SB, B)
    return pl.pallas_call(
        _add_kernel,
        grid=grid,
        in_specs=[
            pl.BlockSpec((1, SB, D), lambda s, b: (b, s, 0)),
            pl.BlockSpec((SB, D), lambda s, b: (s, 0)),
        ],
        out_specs=pl.BlockSpec((1, SB, D), lambda s, b: (b, s, 0)),
        out_shape=jax.ShapeDtypeStruct(x.shape, x.dtype),
        compiler_params=pltpu.CompilerParams(
            dimension_semantics=("parallel", "parallel"),
        ),
    )(x, pos_table[:S])
